# Initial kernel scaffold; baseline (speedup 1.0000x reference)
import jax
import jax.numpy as jnp
from jax.experimental import pallas as pl


def kernel(x, edge_index, We1, be1, We2, be2, We3, be3, We4, be4, Wn1, bn1, Wn2, bn2, Wn3, bn3):
    raise NotImplementedError("write your pallas kernel here")


# speedup vs baseline: 3.9573x; 3.9573x over previous
import functools

import jax
import jax.numpy as jnp
from jax import lax
from jax.experimental import pallas as pl
from jax.experimental.pallas import tpu as pltpu
from jax.experimental.pallas import tpu_sc as plsc

N_NODES = 10000
N_EDGES = 640000
NC, NS = 2, 16
NW = NC * NS
WIN = 128
K_WIN = 160
E_PAD = NW * K_WIN * WIN
D = 16
N_PAD = 10240
G = 8
NB = K_WIN // G
B_MLP = 5120

@functools.cache
def _vector_mesh():
    return plsc.VectorSubcoreMesh(core_axis_name="c", subcore_axis_name="s",
                                  num_cores=NC, num_subcores=NS)


def _gather_body(xp_hbm, dst_hbm, src_hbm, xi_hbm, xj_hbm,
                 idxd, idxs, rowsd, rowss, semgd, semgs, semwd, semws):
    wid = lax.axis_index("s") * NC + lax.axis_index("c")
    w0 = wid * K_WIN
    pltpu.sync_copy(dst_hbm.at[pl.ds(w0, K_WIN)], idxd)
    pltpu.sync_copy(src_hbm.at[pl.ds(w0, K_WIN)], idxs)

    @pl.loop(0, NB)
    def _(nb):
        k0 = nb * G
        for b in range(G):
            pltpu.async_copy(xp_hbm.at[idxd.at[k0 + b]], rowsd.at[b], semgd)
        for b in range(G):
            pltpu.async_copy(xp_hbm.at[idxs.at[k0 + b]], rowss.at[b], semgs)
        for b in range(G):
            pltpu.make_async_copy(
                xp_hbm.at[idxd.at[k0 + b]], rowsd.at[b], semgd).wait()
        for b in range(G):
            pltpu.async_copy(
                rowsd.at[b], xi_hbm.at[pl.ds((w0 + k0 + b) * WIN, WIN)], semwd)
        for b in range(G):
            pltpu.make_async_copy(
                xp_hbm.at[idxs.at[k0 + b]], rowss.at[b], semgs).wait()
        for b in range(G):
            pltpu.async_copy(
                rowss.at[b], xj_hbm.at[pl.ds((w0 + k0 + b) * WIN, WIN)], semws)
        for b in range(G):
            pltpu.make_async_copy(
                rowsd.at[b], xi_hbm.at[pl.ds((w0 + k0 + b) * WIN, WIN)],
                semwd).wait()
        for b in range(G):
            pltpu.make_async_copy(
                rowss.at[b], xj_hbm.at[pl.ds((w0 + k0 + b) * WIN, WIN)],
                semws).wait()


@functools.cache
def _sc_gather():
    return pl.kernel(
        _gather_body, mesh=_vector_mesh(),
        compiler_params=pltpu.CompilerParams(use_tc_tiling_on_sc=False),
        out_type=[jax.ShapeDtypeStruct((E_PAD, D), jnp.float32),
                  jax.ShapeDtypeStruct((E_PAD, D), jnp.float32)],
        scratch_types=[
            pltpu.VMEM((K_WIN, WIN), jnp.int32),
            pltpu.VMEM((K_WIN, WIN), jnp.int32),
            pltpu.VMEM((G, WIN, D), jnp.float32),
            pltpu.VMEM((G, WIN, D), jnp.float32),
            pltpu.SemaphoreType.DMA,
            pltpu.SemaphoreType.DMA,
            pltpu.SemaphoreType.DMA,
            pltpu.SemaphoreType.DMA,
        ],
    )


_ZROWS = 64
_ROWS_PER_TILE = N_PAD // NS


def _scatter_body(m_hbm, dst_hbm, out_hbm, idxd, mbuf, zbuf, acc, semld):
    cid = lax.axis_index("c")
    sid = lax.axis_index("s")
    wid = sid * NC + cid
    w0 = wid * K_WIN

    @pl.loop(0, _ZROWS)
    def _(r):
        zbuf[r, :] = jnp.zeros((D,), jnp.float32)

    @pl.loop(0, _ROWS_PER_TILE // _ZROWS)
    def _(i):
        pltpu.sync_copy(
            zbuf, acc.at[pl.ds(sid * _ROWS_PER_TILE + i * _ZROWS, _ZROWS)])

    pltpu.sync_copy(dst_hbm.at[pl.ds(w0, K_WIN)], idxd)
    plsc.subcore_barrier()

    @pl.loop(0, NB)
    def _(nb):
        k0 = nb * G
        for b in range(G):
            pltpu.async_copy(
                m_hbm.at[pl.ds((w0 + k0 + b) * WIN, WIN)], mbuf.at[b], semld)
        for b in range(G):
            pltpu.make_async_copy(
                m_hbm.at[pl.ds((w0 + k0 + b) * WIN, WIN)], mbuf.at[b],
                semld).wait()
            pltpu.sync_copy(mbuf.at[b], acc.at[idxd.at[k0 + b]], add=True)

    plsc.subcore_barrier()
    pltpu.sync_copy(
        acc.at[pl.ds(sid * _ROWS_PER_TILE, _ROWS_PER_TILE)],
        out_hbm.at[cid].at[pl.ds(sid * _ROWS_PER_TILE, _ROWS_PER_TILE)])


@functools.cache
def _sc_scatter():
    return pl.kernel(
        _scatter_body, mesh=_vector_mesh(),
        compiler_params=pltpu.CompilerParams(use_tc_tiling_on_sc=False),
        out_type=jax.ShapeDtypeStruct((NC, N_PAD, D), jnp.float32),
        scratch_types=[
            pltpu.VMEM((K_WIN, WIN), jnp.int32),
            pltpu.VMEM((G, WIN, D), jnp.float32),
            pltpu.VMEM((_ZROWS, D), jnp.float32),
            pltpu.VMEM_SHARED((N_PAD, D), jnp.float32),
            pltpu.SemaphoreType.DMA,
        ],
    )


def _mlp_body(xi_ref, xj_ref, w1d_ref, w1s_ref, b1_ref, w2_ref, b2_ref,
              w3_ref, b3_ref, w4_ref, b4_ref, m_ref):
    bf = jnp.bfloat16
    f32 = jnp.float32
    h = jnp.dot(xi_ref[...].astype(bf), w1d_ref[...],
                preferred_element_type=f32)
    h = h + jnp.dot(xj_ref[...].astype(bf), w1s_ref[...],
                    preferred_element_type=f32)
    h = jnp.maximum(h + b1_ref[...], 0.0)
    h = jnp.maximum(
        jnp.dot(h.astype(bf), w2_ref[...], preferred_element_type=f32)
        + b2_ref[...], 0.0)
    h = jnp.maximum(
        jnp.dot(h.astype(bf), w3_ref[...], preferred_element_type=f32)
        + b3_ref[...], 0.0)
    m_ref[...] = (jnp.dot(h.astype(bf), w4_ref[...],
                          preferred_element_type=f32) + b4_ref[...])


def _full(shape):
    return pl.BlockSpec(shape, lambda i: tuple(0 for _ in shape))


_tc_mlp = pl.pallas_call(
    _mlp_body,
    grid=(E_PAD // B_MLP,),
    in_specs=[
        pl.BlockSpec((B_MLP, D), lambda i: (i, 0)),
        pl.BlockSpec((B_MLP, D), lambda i: (i, 0)),
        _full((D, 300)), _full((D, 300)), _full((1, 300)),
        _full((300, 300)), _full((1, 300)),
        _full((300, 300)), _full((1, 300)),
        _full((300, D)), _full((1, D)),
    ],
    out_specs=pl.BlockSpec((B_MLP, D), lambda i: (i, 0)),
    out_shape=jax.ShapeDtypeStruct((E_PAD, D), jnp.float32),
)


def _node_body(x_ref, p_ref, wn1_ref, bn1_ref, wn2_ref, bn2_ref,
               wn3_ref, bn3_ref, out_ref):
    p = p_ref[...]
    ea = p[0, :N_NODES, :2] + p[1, :N_NODES, :2]
    z = jnp.concatenate([x_ref[...], ea], axis=1)
    h = jnp.maximum(jnp.dot(z, wn1_ref[...],
                            preferred_element_type=jnp.float32)
                    + bn1_ref[...], 0.0)
    h = jnp.maximum(jnp.dot(h, wn2_ref[...],
                            preferred_element_type=jnp.float32)
                    + bn2_ref[...], 0.0)
    out_ref[...] = (jnp.dot(h, wn3_ref[...],
                            preferred_element_type=jnp.float32)
                    + bn3_ref[...])


_tc_node = pl.pallas_call(
    _node_body,
    out_shape=jax.ShapeDtypeStruct((N_NODES, 2), jnp.float32),
)


def kernel(x, edge_index, We1, be1, We2, be2, We3, be3, We4, be4,
           Wn1, bn1, Wn2, bn2, Wn3, bn3):
    f32 = jnp.float32
    bf = jnp.bfloat16
    xp = jnp.pad(x.astype(f32), ((0, 0), (0, D - 3)))
    src = edge_index[0]
    dst = edge_index[1]
    pad = E_PAD - N_EDGES
    dstp = jnp.concatenate(
        [dst, jnp.full((pad,), N_NODES, jnp.int32)]).reshape(E_PAD // WIN, WIN)
    srcp = jnp.concatenate(
        [src, jnp.zeros((pad,), jnp.int32)]).reshape(E_PAD // WIN, WIN)

    xi, xj = _sc_gather()(xp, dstp, srcp)

    w1d = jnp.pad(We1[:3], ((0, D - 3), (0, 0))).astype(bf)
    w1s = jnp.pad(We1[3:], ((0, D - 3), (0, 0))).astype(bf)
    w4 = jnp.pad(We4, ((0, 0), (0, D - 2))).astype(bf)
    b4 = jnp.pad(be4, (0, D - 2)).reshape(1, D)
    m = _tc_mlp(xi, xj, w1d, w1s, be1.reshape(1, 300),
                We2.astype(bf), be2.reshape(1, 300),
                We3.astype(bf), be3.reshape(1, 300), w4, b4)

    parts = _sc_scatter()(m, dstp)

    return _tc_node(x, parts, Wn1, bn1.reshape(1, 32),
                    Wn2, bn2.reshape(1, 32), Wn3, bn3.reshape(1, 2))


# speedup vs baseline: 5.2994x; 1.3392x over previous
import functools

import jax
import jax.numpy as jnp
from jax import lax
from jax.experimental import pallas as pl
from jax.experimental.pallas import tpu as pltpu
from jax.experimental.pallas import tpu_sc as plsc

N_NODES = 10000
N_EDGES = 640000
NC, NS = 2, 16
NW = NC * NS
WIN = 128
K_WIN = 160
E_PAD = NW * K_WIN * WIN
D = 16
N_PAD = 10240
G = 8
NB = K_WIN // G
B_MLP = 5120

@functools.cache
def _vector_mesh():
    return plsc.VectorSubcoreMesh(core_axis_name="c", subcore_axis_name="s",
                                  num_cores=NC, num_subcores=NS)


def _gather_body(xp_hbm, dst_hbm, src_hbm, xi_hbm, xj_hbm,
                 idxd, idxs, rowsd, rowss, semgd, semgs, semwd, semws):
    wid = lax.axis_index("s") * NC + lax.axis_index("c")
    w0 = wid * K_WIN
    pltpu.sync_copy(dst_hbm.at[pl.ds(w0, K_WIN)], idxd)
    pltpu.sync_copy(src_hbm.at[pl.ds(w0, K_WIN)], idxs)

    @pl.loop(0, NB)
    def _(nb):
        k0 = nb * G
        for b in range(G):
            pltpu.async_copy(xp_hbm.at[idxd.at[k0 + b]], rowsd.at[b], semgd)
        for b in range(G):
            pltpu.async_copy(xp_hbm.at[idxs.at[k0 + b]], rowss.at[b], semgs)
        for b in range(G):
            pltpu.make_async_copy(
                xp_hbm.at[idxd.at[k0 + b]], rowsd.at[b], semgd).wait()
        for b in range(G):
            pltpu.async_copy(
                rowsd.at[b], xi_hbm.at[pl.ds((w0 + k0 + b) * WIN, WIN)], semwd)
        for b in range(G):
            pltpu.make_async_copy(
                xp_hbm.at[idxs.at[k0 + b]], rowss.at[b], semgs).wait()
        for b in range(G):
            pltpu.async_copy(
                rowss.at[b], xj_hbm.at[pl.ds((w0 + k0 + b) * WIN, WIN)], semws)
        for b in range(G):
            pltpu.make_async_copy(
                rowsd.at[b], xi_hbm.at[pl.ds((w0 + k0 + b) * WIN, WIN)],
                semwd).wait()
        for b in range(G):
            pltpu.make_async_copy(
                rowss.at[b], xj_hbm.at[pl.ds((w0 + k0 + b) * WIN, WIN)],
                semws).wait()


@functools.cache
def _sc_gather():
    return pl.kernel(
        _gather_body, mesh=_vector_mesh(),
        compiler_params=pltpu.CompilerParams(use_tc_tiling_on_sc=False),
        out_type=[jax.ShapeDtypeStruct((E_PAD, D), jnp.float32),
                  jax.ShapeDtypeStruct((E_PAD, D), jnp.float32)],
        scratch_types=[
            pltpu.VMEM((K_WIN, WIN), jnp.int32),
            pltpu.VMEM((K_WIN, WIN), jnp.int32),
            pltpu.VMEM((G, WIN, D), jnp.float32),
            pltpu.VMEM((G, WIN, D), jnp.float32),
            pltpu.SemaphoreType.DMA,
            pltpu.SemaphoreType.DMA,
            pltpu.SemaphoreType.DMA,
            pltpu.SemaphoreType.DMA,
        ],
    )


_ZROWS = 64
_ROWS_PER_TILE = N_PAD // NS


def _scatter_body(m_hbm, dst_hbm, out_hbm, idxd, mbuf, zbuf, acc, semld):
    cid = lax.axis_index("c")
    sid = lax.axis_index("s")
    wid = sid * NC + cid
    w0 = wid * K_WIN

    @pl.loop(0, _ZROWS)
    def _(r):
        zbuf[r, :] = jnp.zeros((D,), jnp.float32)

    @pl.loop(0, _ROWS_PER_TILE // _ZROWS)
    def _(i):
        pltpu.sync_copy(
            zbuf, acc.at[pl.ds(sid * _ROWS_PER_TILE + i * _ZROWS, _ZROWS)])

    pltpu.sync_copy(dst_hbm.at[pl.ds(w0, K_WIN)], idxd)
    plsc.subcore_barrier()

    @pl.loop(0, NB)
    def _(nb):
        k0 = nb * G
        for b in range(G):
            pltpu.async_copy(
                m_hbm.at[pl.ds((w0 + k0 + b) * WIN, WIN)], mbuf.at[b], semld)
        for b in range(G):
            pltpu.make_async_copy(
                m_hbm.at[pl.ds((w0 + k0 + b) * WIN, WIN)], mbuf.at[b],
                semld).wait()
            pltpu.sync_copy(mbuf.at[b], acc.at[idxd.at[k0 + b]], add=True)

    plsc.subcore_barrier()
    pltpu.sync_copy(
        acc.at[pl.ds(sid * _ROWS_PER_TILE, _ROWS_PER_TILE)],
        out_hbm.at[cid].at[pl.ds(sid * _ROWS_PER_TILE, _ROWS_PER_TILE)])


@functools.cache
def _sc_scatter():
    return pl.kernel(
        _scatter_body, mesh=_vector_mesh(),
        compiler_params=pltpu.CompilerParams(use_tc_tiling_on_sc=False),
        out_type=jax.ShapeDtypeStruct((NC, N_PAD, D), jnp.float32),
        scratch_types=[
            pltpu.VMEM((K_WIN, WIN), jnp.int32),
            pltpu.VMEM((G, WIN, D), jnp.float32),
            pltpu.VMEM((_ZROWS, D), jnp.float32),
            pltpu.VMEM_SHARED((N_PAD, D), jnp.float32),
            pltpu.SemaphoreType.DMA,
        ],
    )


_PACK = 128 // D
_RX = E_PAD // _PACK
_RB = B_MLP // _PACK


def _mlp_body(xi_ref, xj_ref, w1d_ref, w1s_ref, b1_ref, w2_ref, b2_ref,
              w3_ref, b3_ref, w4_ref, b4_ref, m_ref):
    bf = jnp.bfloat16
    f32 = jnp.float32
    pi = xi_ref[...].astype(bf)
    pj = xj_ref[...].astype(bf)
    m = jnp.broadcast_to(b4_ref[...], (_RB, 128))
    for g in range(_PACK):
        h = jnp.dot(pi, w1d_ref[g], preferred_element_type=f32)
        h = h + jnp.dot(pj, w1s_ref[g], preferred_element_type=f32)
        h = jnp.maximum(h + b1_ref[...], 0.0)
        h = jnp.maximum(
            jnp.dot(h.astype(bf), w2_ref[...], preferred_element_type=f32)
            + b2_ref[...], 0.0)
        h = jnp.maximum(
            jnp.dot(h.astype(bf), w3_ref[...], preferred_element_type=f32)
            + b3_ref[...], 0.0)
        m = m + jnp.dot(h.astype(bf), w4_ref[g], preferred_element_type=f32)
    m_ref[...] = m


def _full(shape):
    return pl.BlockSpec(shape, lambda i: tuple(0 for _ in shape))


_tc_mlp = pl.pallas_call(
    _mlp_body,
    grid=(E_PAD // B_MLP,),
    in_specs=[
        pl.BlockSpec((_RB, 128), lambda i: (i, 0)),
        pl.BlockSpec((_RB, 128), lambda i: (i, 0)),
        _full((_PACK, 128, 300)), _full((_PACK, 128, 300)), _full((1, 300)),
        _full((300, 300)), _full((1, 300)),
        _full((300, 300)), _full((1, 300)),
        _full((_PACK, 300, 128)), _full((1, 128)),
    ],
    out_specs=pl.BlockSpec((_RB, 128), lambda i: (i, 0)),
    out_shape=jax.ShapeDtypeStruct((_RX, 128), jnp.float32),
)


def _node_body(x_ref, p_ref, wn1_ref, bn1_ref, wn2_ref, bn2_ref,
               wn3_ref, bn3_ref, out_ref):
    p = p_ref[...]
    ea = p[0, :N_NODES, :2] + p[1, :N_NODES, :2]
    z = jnp.concatenate([x_ref[...], ea], axis=1)
    h = jnp.maximum(jnp.dot(z, wn1_ref[...],
                            preferred_element_type=jnp.float32)
                    + bn1_ref[...], 0.0)
    h = jnp.maximum(jnp.dot(h, wn2_ref[...],
                            preferred_element_type=jnp.float32)
                    + bn2_ref[...], 0.0)
    out_ref[...] = (jnp.dot(h, wn3_ref[...],
                            preferred_element_type=jnp.float32)
                    + bn3_ref[...])


_tc_node = pl.pallas_call(
    _node_body,
    out_shape=jax.ShapeDtypeStruct((N_NODES, 2), jnp.float32),
)


def kernel(x, edge_index, We1, be1, We2, be2, We3, be3, We4, be4,
           Wn1, bn1, Wn2, bn2, Wn3, bn3):
    f32 = jnp.float32
    bf = jnp.bfloat16
    xp = jnp.pad(x.astype(f32), ((0, 0), (0, D - 3)))
    src = edge_index[0]
    dst = edge_index[1]
    pad = E_PAD - N_EDGES
    dstp = jnp.concatenate(
        [dst, jnp.full((pad,), N_NODES, jnp.int32)]).reshape(E_PAD // WIN, WIN)
    srcp = jnp.concatenate(
        [src, jnp.zeros((pad,), jnp.int32)]).reshape(E_PAD // WIN, WIN)

    xi, xj = _sc_gather()(xp, dstp, srcp)

    w1d_pad = jnp.pad(We1[:3], ((0, D - 3), (0, 0)))
    w1s_pad = jnp.pad(We1[3:], ((0, D - 3), (0, 0)))
    eye = jnp.eye(_PACK, dtype=f32)
    w1d_stack = jnp.einsum("gh,kn->ghkn", eye, w1d_pad).reshape(
        _PACK, 128, 300).astype(bf)
    w1s_stack = jnp.einsum("gh,kn->ghkn", eye, w1s_pad).reshape(
        _PACK, 128, 300).astype(bf)
    w4_pad = jnp.pad(We4, ((0, 0), (0, D - 2)))
    w4_stack = jnp.einsum("nk,gh->nghk", w4_pad, eye).reshape(
        300, _PACK, 128).transpose(1, 0, 2).astype(bf)
    b4 = jnp.tile(jnp.pad(be4, (0, D - 2)), _PACK).reshape(1, 128)
    m_packed = _tc_mlp(xi.reshape(_RX, 128), xj.reshape(_RX, 128),
                       w1d_stack, w1s_stack, be1.reshape(1, 300),
                       We2.astype(bf), be2.reshape(1, 300),
                       We3.astype(bf), be3.reshape(1, 300), w4_stack, b4)

    parts = _sc_scatter()(m_packed.reshape(E_PAD, D), dstp)

    return _tc_node(x, parts, Wn1, bn1.reshape(1, 32),
                    Wn2, bn2.reshape(1, 32), Wn3, bn3.reshape(1, 2))


# speedup vs baseline: 7.3612x; 1.3891x over previous
import functools

import jax
import jax.numpy as jnp
from jax import lax
from jax.experimental import pallas as pl
from jax.experimental.pallas import tpu as pltpu
from jax.experimental.pallas import tpu_sc as plsc

N_NODES = 10000
N_EDGES = 640000
NC, NS = 2, 16
NW = NC * NS
WIN = 128
CH = 4
K_WIN = 160 // CH
E_CH = NW * K_WIN * WIN
E_PAD = E_CH * CH
WR_CH = E_CH // WIN
D = 16
N_PAD = 10240
G = 4
NB = K_WIN // G
B_MLP = 5120

@functools.cache
def _vector_mesh():
    return plsc.VectorSubcoreMesh(core_axis_name="c", subcore_axis_name="s",
                                  num_cores=NC, num_subcores=NS)


def _gather_body(xp_hbm, dst_hbm, src_hbm, xi_hbm, xj_hbm,
                 idxd, idxs, rowsd, rowss, semgd, semgs, semwd, semws):
    wid = lax.axis_index("s") * NC + lax.axis_index("c")
    w0 = wid * K_WIN
    pltpu.sync_copy(dst_hbm.at[pl.ds(w0, K_WIN)], idxd)
    pltpu.sync_copy(src_hbm.at[pl.ds(w0, K_WIN)], idxs)

    @pl.loop(0, NB)
    def _(nb):
        k0 = nb * G
        for b in range(G):
            pltpu.async_copy(xp_hbm.at[idxd.at[k0 + b]], rowsd.at[b], semgd)
        for b in range(G):
            pltpu.async_copy(xp_hbm.at[idxs.at[k0 + b]], rowss.at[b], semgs)
        for b in range(G):
            pltpu.make_async_copy(
                xp_hbm.at[idxd.at[k0 + b]], rowsd.at[b], semgd).wait()
        for b in range(G):
            pltpu.async_copy(
                rowsd.at[b], xi_hbm.at[pl.ds((w0 + k0 + b) * WIN, WIN)], semwd)
        for b in range(G):
            pltpu.make_async_copy(
                xp_hbm.at[idxs.at[k0 + b]], rowss.at[b], semgs).wait()
        for b in range(G):
            pltpu.async_copy(
                rowss.at[b], xj_hbm.at[pl.ds((w0 + k0 + b) * WIN, WIN)], semws)
        for b in range(G):
            pltpu.make_async_copy(
                rowsd.at[b], xi_hbm.at[pl.ds((w0 + k0 + b) * WIN, WIN)],
                semwd).wait()
        for b in range(G):
            pltpu.make_async_copy(
                rowss.at[b], xj_hbm.at[pl.ds((w0 + k0 + b) * WIN, WIN)],
                semws).wait()


@functools.cache
def _sc_gather():
    return pl.kernel(
        _gather_body, mesh=_vector_mesh(),
        compiler_params=pltpu.CompilerParams(use_tc_tiling_on_sc=False),
        out_type=[jax.ShapeDtypeStruct((E_CH, D), jnp.float32),
                  jax.ShapeDtypeStruct((E_CH, D), jnp.float32)],
        scratch_types=[
            pltpu.VMEM((K_WIN, WIN), jnp.int32),
            pltpu.VMEM((K_WIN, WIN), jnp.int32),
            pltpu.VMEM((G, WIN, D), jnp.float32),
            pltpu.VMEM((G, WIN, D), jnp.float32),
            pltpu.SemaphoreType.DMA,
            pltpu.SemaphoreType.DMA,
            pltpu.SemaphoreType.DMA,
            pltpu.SemaphoreType.DMA,
        ],
    )


_ZROWS = 64
_ROWS_PER_TILE = N_PAD // NS


def _scatter_body(m_hbm, dst_hbm, out_hbm, idxd, mbuf, zbuf, acc, semld):
    cid = lax.axis_index("c")
    sid = lax.axis_index("s")
    wid = sid * NC + cid
    w0 = wid * K_WIN

    @pl.loop(0, _ZROWS)
    def _(r):
        zbuf[r, :] = jnp.zeros((D,), jnp.float32)

    @pl.loop(0, _ROWS_PER_TILE // _ZROWS)
    def _(i):
        pltpu.sync_copy(
            zbuf, acc.at[pl.ds(sid * _ROWS_PER_TILE + i * _ZROWS, _ZROWS)])

    pltpu.sync_copy(dst_hbm.at[pl.ds(w0, K_WIN)], idxd)
    plsc.subcore_barrier()

    @pl.loop(0, NB)
    def _(nb):
        k0 = nb * G
        for b in range(G):
            pltpu.async_copy(
                m_hbm.at[pl.ds((w0 + k0 + b) * WIN, WIN)], mbuf.at[b], semld)
        for b in range(G):
            pltpu.make_async_copy(
                m_hbm.at[pl.ds((w0 + k0 + b) * WIN, WIN)], mbuf.at[b],
                semld).wait()
            pltpu.sync_copy(mbuf.at[b], acc.at[idxd.at[k0 + b]], add=True)

    plsc.subcore_barrier()
    pltpu.sync_copy(
        acc.at[pl.ds(sid * _ROWS_PER_TILE, _ROWS_PER_TILE)],
        out_hbm.at[cid].at[pl.ds(sid * _ROWS_PER_TILE, _ROWS_PER_TILE)])


@functools.cache
def _sc_scatter():
    return pl.kernel(
        _scatter_body, mesh=_vector_mesh(),
        compiler_params=pltpu.CompilerParams(use_tc_tiling_on_sc=False),
        out_type=jax.ShapeDtypeStruct((NC, N_PAD, D), jnp.float32),
        scratch_types=[
            pltpu.VMEM((K_WIN, WIN), jnp.int32),
            pltpu.VMEM((G, WIN, D), jnp.float32),
            pltpu.VMEM((_ZROWS, D), jnp.float32),
            pltpu.VMEM_SHARED((N_PAD, D), jnp.float32),
            pltpu.SemaphoreType.DMA,
        ],
    )


_PACK = 128 // D
_RX = E_CH // _PACK
_RB = B_MLP // _PACK


_GW = 384


def _mlp_body(xi_ref, xj_ref, w1_ref, b1_ref, w2_ref, b2_ref,
              w3_ref, b3_ref, w4_ref, b4_ref, m_ref):
    bf = jnp.bfloat16
    f32 = jnp.float32
    pij = jnp.concatenate([xi_ref[...], xj_ref[...]], axis=1).astype(bf)
    hwide = jnp.dot(pij, w1_ref[...], preferred_element_type=f32)
    hwide = jnp.maximum(hwide.astype(bf) + b1_ref[...], 0)
    h = jnp.concatenate(
        [hwide[:, g * _GW:g * _GW + 300] for g in range(_PACK)], axis=0)
    h = jnp.maximum(
        jnp.dot(h, w2_ref[...], preferred_element_type=f32).astype(bf)
        + b2_ref[...], 0)
    h = jnp.maximum(
        jnp.dot(h, w3_ref[...], preferred_element_type=f32).astype(bf)
        + b3_ref[...], 0)
    m = jnp.broadcast_to(b4_ref[...], (_RB, 128))
    for g in range(_PACK):
        m = m + jnp.dot(h[g * _RB:(g + 1) * _RB],
                        w4_ref[g], preferred_element_type=f32)
    m_ref[...] = m


def _full(shape):
    return pl.BlockSpec(shape, lambda i: tuple(0 for _ in shape))


_tc_mlp = pl.pallas_call(
    _mlp_body,
    grid=(E_CH // B_MLP,),
    in_specs=[
        pl.BlockSpec((_RB, 128), lambda i: (i, 0)),
        pl.BlockSpec((_RB, 128), lambda i: (i, 0)),
        _full((256, _PACK * _GW)), _full((1, _PACK * _GW)),
        _full((300, 300)), _full((1, 300)),
        _full((300, 300)), _full((1, 300)),
        _full((_PACK, 300, 128)), _full((1, 128)),
    ],
    out_specs=pl.BlockSpec((_RB, 128), lambda i: (i, 0)),
    out_shape=jax.ShapeDtypeStruct((_RX, 128), jnp.float32),
)


def _node_body(*refs):
    x_ref = refs[0]
    part_refs = refs[1:1 + CH]
    wn1_ref, bn1_ref, wn2_ref, bn2_ref, wn3_ref, bn3_ref = refs[1 + CH:-1]
    out_ref = refs[-1]
    ea = jnp.zeros((N_NODES, 2), jnp.float32)
    for p_ref in part_refs:
        p = p_ref[...]
        ea = ea + p[0, :N_NODES, :2] + p[1, :N_NODES, :2]
    z = jnp.concatenate([x_ref[...], ea], axis=1)
    h = jnp.maximum(jnp.dot(z, wn1_ref[...],
                            preferred_element_type=jnp.float32)
                    + bn1_ref[...], 0.0)
    h = jnp.maximum(jnp.dot(h, wn2_ref[...],
                            preferred_element_type=jnp.float32)
                    + bn2_ref[...], 0.0)
    out_ref[...] = (jnp.dot(h, wn3_ref[...],
                            preferred_element_type=jnp.float32)
                    + bn3_ref[...])


_tc_node = pl.pallas_call(
    _node_body,
    out_shape=jax.ShapeDtypeStruct((N_NODES, 2), jnp.float32),
)


def kernel(x, edge_index, We1, be1, We2, be2, We3, be3, We4, be4,
           Wn1, bn1, Wn2, bn2, Wn3, bn3):
    f32 = jnp.float32
    bf = jnp.bfloat16
    xp = jnp.pad(x.astype(f32), ((0, 0), (0, D - 3)))
    src = edge_index[0]
    dst = edge_index[1]
    pad = E_PAD - N_EDGES
    dstp = jnp.concatenate(
        [dst, jnp.full((pad,), N_NODES, jnp.int32)]).reshape(E_PAD // WIN, WIN)
    srcp = jnp.concatenate(
        [src, jnp.zeros((pad,), jnp.int32)]).reshape(E_PAD // WIN, WIN)

    w1d_pad = jnp.pad(We1[:3], ((0, D - 3), (0, _GW - 300)))
    w1s_pad = jnp.pad(We1[3:], ((0, D - 3), (0, _GW - 300)))
    eye = jnp.eye(_PACK, dtype=f32)
    w1d_cat = jnp.einsum("gh,kn->hkgn", eye, w1d_pad).reshape(
        128, _PACK * _GW)
    w1s_cat = jnp.einsum("gh,kn->hkgn", eye, w1s_pad).reshape(
        128, _PACK * _GW)
    w1cat = jnp.concatenate([w1d_cat, w1s_cat], axis=0).astype(bf)
    b1cat = jnp.tile(jnp.pad(be1, (0, _GW - 300)), _PACK).reshape(
        1, _PACK * _GW).astype(bf)
    w4_pad = jnp.pad(We4, ((0, 0), (0, D - 2)))
    w4_stack = jnp.einsum("nk,gh->nghk", w4_pad, eye).reshape(
        300, _PACK, 128).transpose(1, 0, 2).astype(bf)
    b4 = jnp.tile(jnp.pad(be4, (0, D - 2)), _PACK).reshape(1, 128)

    all_parts = []
    for c in range(CH):
        if c < CH - 1:
            dst_c = lax.slice_in_dim(
                dst, c * E_CH, (c + 1) * E_CH).reshape(WR_CH, WIN)
            src_c = lax.slice_in_dim(
                src, c * E_CH, (c + 1) * E_CH).reshape(WR_CH, WIN)
        else:
            dst_c = lax.slice_in_dim(dstp, c * WR_CH, (c + 1) * WR_CH, axis=0)
            src_c = lax.slice_in_dim(srcp, c * WR_CH, (c + 1) * WR_CH, axis=0)
        xi, xj = _sc_gather()(xp, dst_c, src_c)
        m_packed = _tc_mlp(xi.reshape(_RX, 128), xj.reshape(_RX, 128),
                           w1cat, b1cat,
                           We2.astype(bf), be2.reshape(1, 300).astype(bf),
                           We3.astype(bf), be3.reshape(1, 300).astype(bf),
                           w4_stack, b4)
        all_parts.append(_sc_scatter()(m_packed.reshape(E_CH, D), dst_c))

    return _tc_node(x, *all_parts, Wn1, bn1.reshape(1, 32),
                    Wn2, bn2.reshape(1, 32), Wn3, bn3.reshape(1, 2))
